# baseline (speedup 1.0000x reference)
import jax
import jax.numpy as jnp
from jax import lax
from jax.experimental import pallas as pl
from jax.experimental.pallas import tpu as pltpu
from jax.experimental.pallas import tpu_sc as plsc

VOCAB_SIZE = 1000000
SEQ_LEN = 200
EMBED_DIM = 32
BATCH = 4096

NC = 2
NS = 16
NW = NC * NS

B_FLAT = BATCH * SEQ_LEN
ROWS_PER_W = B_FLAT // NW
G = 128
C = 1024
CG = C // G
NCHUNK = ROWS_PER_W // C
CB = C // 4


def _sc_body(idx_hbm, tok_hbm, pos_hbm, out4_hbm,
             idx_v, bounce_v, out_v, pos_v, sem):
    wid = lax.axis_index("s") * NC + lax.axis_index("c")

    pltpu.sync_copy(pos_hbm, pos_v)

    @pl.loop(0, NCHUNK)
    def _chunk(c):
        crow = wid * NCHUNK + c
        pltpu.sync_copy(idx_hbm.at[pl.ds(crow * CG, CG)], idx_v)

        descs = [
            pltpu.async_copy(
                tok_hbm.at[idx_v.at[j]],
                bounce_v.at[pl.ds(j * G, G)],
                sem,
            )
            for j in range(CG)
        ]
        for d in descs:
            d.wait()

        chunk_base = c * C
        p_init = tuple(
            lax.rem(chunk_base + 4 * (j % 2) * G + (j // 2), SEQ_LEN)
            for j in range(CG)
        )

        @pl.loop(0, G, init_carry=p_init, unroll=2)
        def _row(t, ps):
            out = []
            for j in range(CG):
                q, half = j // 2, j % 2
                p = ps[j]
                for h in range(2):
                    out_v[half * G + t, pl.ds(q * 32 + h * 16, 16)] = (
                        bounce_v[j * G + t, pl.ds(h * 16, 16)]
                        + pos_v[p, pl.ds(h * 16, 16)]
                    )
                p = p + 4
                out.append(jnp.where(p >= SEQ_LEN, p - SEQ_LEN, p))
            return tuple(out)

        pltpu.sync_copy(out_v, out4_hbm.at[pl.ds(wid * (ROWS_PER_W // 4) + c * CB, CB)])


@jax.jit
def _sc_embed(idx, token_table, position_table):
    mesh = plsc.VectorSubcoreMesh(
        core_axis_name="c", subcore_axis_name="s", num_cores=NC, num_subcores=NS
    )
    return pl.kernel(
        _sc_body,
        out_type=jax.ShapeDtypeStruct((B_FLAT // 4, 128), jnp.float32),
        mesh=mesh,
        scratch_types=[
            pltpu.VMEM((CG, G), jnp.int32),
            pltpu.VMEM((C, EMBED_DIM), jnp.float32),
            pltpu.VMEM((CB, 128), jnp.float32),
            pltpu.VMEM((SEQ_LEN, EMBED_DIM), jnp.float32),
            pltpu.SemaphoreType.DMA,
        ],
        compiler_params=pltpu.CompilerParams(use_tc_tiling_on_sc=False),
    )(idx, token_table, position_table)


def kernel(inputs, token_table, position_table):
    idx = (
        inputs.astype(jnp.int32)
        .reshape(NW, NCHUNK, 2, G, 4)
        .transpose(0, 1, 4, 2, 3)
        .reshape(NW * NCHUNK * CG, G)
    )
    out = _sc_embed(idx, token_table, position_table)
    return out.reshape(BATCH, SEQ_LEN, EMBED_DIM)
